# baseline (speedup 1.0000x reference)
import functools

import jax
import jax.numpy as jnp
from jax import lax
from jax.experimental import pallas as pl
from jax.experimental.pallas import tpu as pltpu
from jax.experimental.pallas import tpu_sc as plsc

_SIDE = 32
_LANES = 16


def _mask_reduce_body(m_ref, scale_ref):
    i = pl.program_id(0)
    part = jnp.sum(m_ref[0, 0], axis=0, keepdims=True)

    @pl.when(i == 0)
    def _():
        scale_ref[...] = part

    @pl.when(i > 0)
    def _():
        scale_ref[...] = scale_ref[...] + part

    @pl.when(i == pl.num_programs(0) - 1)
    def _():
        scale_ref[...] = jnp.where(scale_ref[...] == 0.0, 1.0, 0.0)


def _make_sc_kernel(B, T, C, S, rows_per_blk):
    mesh = plsc.VectorSubcoreMesh(core_axis_name="c", subcore_axis_name="s")
    info = plsc.get_sparse_core_info()
    nc, ns = info.num_cores, info.num_subcores
    nw = nc * ns
    rows_per_w = (B * C) // nw
    n_blk = rows_per_w // rows_per_blk
    blk_elems = rows_per_blk * S

    @functools.partial(
        pl.kernel,
        out_type=jax.ShapeDtypeStruct((B * C * S,), jnp.float32),
        mesh=mesh,
        compiler_params=pltpu.CompilerParams(needs_layout_passes=False),
        scratch_types=[
            pltpu.VMEM((S,), jnp.int32),
            pltpu.VMEM((S,), jnp.float32),
            pltpu.VMEM((blk_elems,), jnp.float32),
            pltpu.VMEM((blk_elems,), jnp.float32),
        ],
    )
    def sc_kernel(d_hbm, scale_hbm, poi_hbm, out_hbm, poi_v, scale_v, in_v, out_v):
        wid = lax.axis_index("s") * nc + lax.axis_index("c")
        row0 = wid * rows_per_w

        pltpu.sync_copy(poi_hbm, poi_v)
        pltpu.sync_copy(scale_hbm, scale_v)

        def block_body(blk, carry):
            row_base = row0 + blk * rows_per_blk
            b = row_base // C
            c0 = row_base % C
            src_base = ((b * T + (T - 1)) * C + c0) * S
            pltpu.sync_copy(d_hbm.at[pl.ds(src_base, blk_elems)], in_v)

            def row_body(r, carry2):
                rbase = jnp.full((_LANES,), r * S, jnp.int32)

                def chunk_body(k, carry3):
                    sl = pl.ds(r * S + k * _LANES, _LANES)
                    idx = poi_v[pl.ds(k * _LANES, _LANES)]
                    g = plsc.load_gather(in_v, [rbase + idx])
                    out_v[sl] = in_v[sl] + scale_v[pl.ds(k * _LANES, _LANES)] * g
                    return carry3

                return lax.fori_loop(0, S // _LANES, chunk_body, carry2)

            lax.fori_loop(0, rows_per_blk, row_body, 0)
            pltpu.sync_copy(out_v, out_hbm.at[pl.ds(row_base * S, blk_elems)])
            return carry

        lax.fori_loop(0, n_blk, block_body, 0)

    return sc_kernel


def kernel(d, m, weight, poi_index):
    B, T, C, S = d.shape

    scale = pl.pallas_call(
        _mask_reduce_body,
        grid=(B,),
        in_specs=[pl.BlockSpec((1, 1, C, S), lambda i: (i, T - 1, 0, 0))],
        out_specs=pl.BlockSpec((1, S), lambda i: (0, 0)),
        out_shape=jax.ShapeDtypeStruct((1, S), jnp.float32),
    )(m)

    sc = _make_sc_kernel(B, T, C, S, rows_per_blk=32)
    out = sc(d.reshape(-1), scale.reshape(S), poi_index.astype(jnp.int32))
    return out.reshape(B, C, _SIDE, _SIDE)


# speedup vs baseline: 1.3626x; 1.3626x over previous
import functools

import jax
import jax.numpy as jnp
from jax import lax
from jax.experimental import pallas as pl
from jax.experimental.pallas import tpu as pltpu
from jax.experimental.pallas import tpu_sc as plsc

_SIDE = 32
_LANES = 16


def _mask_reduce_body(m_ref, scale_ref):
    i = pl.program_id(0)
    part = jnp.sum(m_ref[0, 0], axis=0, keepdims=True)

    @pl.when(i == 0)
    def _():
        scale_ref[...] = part

    @pl.when(i > 0)
    def _():
        scale_ref[...] = scale_ref[...] + part

    @pl.when(i == pl.num_programs(0) - 1)
    def _():
        scale_ref[...] = jnp.where(scale_ref[...] == 0.0, 1.0, 0.0)


def _make_sc_kernel(B, T, C, S):
    mesh = plsc.VectorSubcoreMesh(core_axis_name="c", subcore_axis_name="s")
    info = plsc.get_sparse_core_info()
    nc, ns = info.num_cores, info.num_subcores
    nw = nc * ns
    c_per_w = C // (nw // B)
    C_PASS = 64
    SQ = 256

    @functools.partial(
        pl.kernel,
        out_type=jax.ShapeDtypeStruct((B, S, C), jnp.float32),
        mesh=mesh,
        compiler_params=pltpu.CompilerParams(
            needs_layout_passes=False, use_tc_tiling_on_sc=True),
        scratch_types=[
            pltpu.VMEM((S,), jnp.int32),
            pltpu.VMEM((S,), jnp.float32),
            pltpu.VMEM((C_PASS, S), jnp.float32),
            pltpu.VMEM((SQ, 2 * C_PASS), jnp.float32),
        ],
    )
    def sc_kernel(d_hbm, scale_hbm, poi_hbm, out_hbm, poi_v, scale_v, in2d, out2d):
        wid = lax.axis_index("s") * nc + lax.axis_index("c")
        b = wid // (C // c_per_w)
        c_base = (wid % (C // c_per_w)) * c_per_w

        pltpu.sync_copy(poi_hbm, poi_v)
        pltpu.sync_copy(scale_hbm, scale_v)

        cvecs = [lax.iota(jnp.int32, _LANES) + cc * _LANES
                 for cc in range(C_PASS // _LANES)]

        for sq in range(S // SQ):
            for cp in range(c_per_w // C_PASS):
                c0 = c_base + cp * C_PASS
                pltpu.sync_copy(d_hbm.at[b, T - 1, pl.ds(c0, C_PASS), :], in2d)

                def k_body(k, carry, sq=sq, cp=cp):
                    kbase = sq * SQ + k * _LANES
                    svec = scale_v[pl.ds(kbase, _LANES)]
                    pvec = poi_v[pl.ds(kbase, _LANES)]
                    for i in range(_LANES):
                        scol = jnp.full((_LANES,), kbase + i, jnp.int32)
                        pcol = jnp.full((_LANES,), pvec[i], jnp.int32)
                        scv = jnp.full((_LANES,), svec[i], jnp.float32)
                        for cc in range(C_PASS // _LANES):
                            g1 = plsc.load_gather(in2d, [cvecs[cc], scol])
                            g2 = plsc.load_gather(in2d, [cvecs[cc], pcol])
                            out2d[k * _LANES + i,
                                  pl.ds(cp * C_PASS + cc * _LANES, _LANES)] = (
                                g1 + scv * g2)
                    return carry

                lax.fori_loop(0, SQ // _LANES, k_body, 0)

            pltpu.sync_copy(
                out2d, out_hbm.at[b, pl.ds(sq * SQ, SQ), pl.ds(c_base, c_per_w)])

    return sc_kernel


def kernel(d, m, weight, poi_index):
    B, T, C, S = d.shape

    scale = pl.pallas_call(
        _mask_reduce_body,
        grid=(B,),
        in_specs=[pl.BlockSpec((1, 1, C, S), lambda i: (i, T - 1, 0, 0))],
        out_specs=pl.BlockSpec((1, S), lambda i: (0, 0)),
        out_shape=jax.ShapeDtypeStruct((1, S), jnp.float32),
    )(m)

    sc = _make_sc_kernel(B, T, C, S)
    out_t = sc(d, scale.reshape(S), poi_index.astype(jnp.int32))
    return out_t.reshape(B, _SIDE, _SIDE, C).transpose(0, 3, 1, 2)


# speedup vs baseline: 2.0681x; 1.5177x over previous
import functools

import jax
import jax.numpy as jnp
from jax import lax
from jax.experimental import pallas as pl
from jax.experimental.pallas import tpu as pltpu
from jax.experimental.pallas import tpu_sc as plsc

_SIDE = 32
_LANES = 16


def _scale_transpose_body(m_ref, d_ref, scale_ref, dt_ref):
    i = pl.program_id(0)
    part = jnp.sum(m_ref[0, 0], axis=0, keepdims=True)

    @pl.when(i == 0)
    def _():
        scale_ref[...] = part

    @pl.when(i > 0)
    def _():
        scale_ref[...] = scale_ref[...] + part

    @pl.when(i == pl.num_programs(0) - 1)
    def _():
        scale_ref[...] = jnp.where(scale_ref[...] == 0.0, 1.0, 0.0)

    dt_ref[0] = d_ref[0, 0].T


def _make_sc_kernel(B, T, C, S):
    mesh = plsc.VectorSubcoreMesh(core_axis_name="c", subcore_axis_name="s")
    info = plsc.get_sparse_core_info()
    nc, ns = info.num_cores, info.num_subcores
    nw = nc * ns
    halves = nw // B
    s_per_w = S // halves
    SB = 128
    DUMP = SB

    @functools.partial(
        pl.kernel,
        out_type=jax.ShapeDtypeStruct((B, S, C), jnp.float32),
        mesh=mesh,
        compiler_params=pltpu.CompilerParams(
            needs_layout_passes=False, use_tc_tiling_on_sc=True),
        scratch_types=[
            pltpu.VMEM((S,), jnp.int32),
            pltpu.VMEM((S,), jnp.float32),
            pltpu.VMEM((SB + 8, C), jnp.float32),
            pltpu.VMEM((SB, C), jnp.float32),
            pltpu.VMEM((SB + _LANES, ), jnp.int32),
            pltpu.VMEM((SB,), jnp.int32),
            pltpu.SemaphoreType.DMA,
        ],
    )
    def sc_kernel(dt_hbm, scale_hbm, poi_hbm, out_hbm,
                  poi_v, scale_v, slab, rows_v, dsts_v, srcs_v, sem):
        wid = lax.axis_index("s") * nc + lax.axis_index("c")
        b = wid // halves
        s_base = (wid % halves) * s_per_w

        pltpu.sync_copy(poi_hbm, poi_v)
        pltpu.sync_copy(scale_hbm, scale_v)
        iota = lax.iota(jnp.int32, _LANES)

        for chunk in range(s_per_w // SB):
            s0 = s_base + chunk * SB

            for t in range((SB + _LANES) // _LANES):
                dsts_v[pl.ds(t * _LANES, _LANES)] = jnp.full(
                    (_LANES,), DUMP, jnp.int32)
            for t in range(SB // _LANES):
                srcs_v[pl.ds(t * _LANES, _LANES)] = jnp.zeros(
                    (_LANES,), jnp.int32)

            off = jnp.int32(0)
            for k8 in range(SB // _LANES):
                sv = scale_v[pl.ds(s0 + k8 * _LANES, _LANES)]
                pv = poi_v[pl.ds(s0 + k8 * _LANES, _LANES)]
                msk = sv != 0.0
                plsc.store_compressed(
                    dsts_v.at[pl.ds(off, _LANES)], iota + (k8 * _LANES),
                    mask=msk)
                plsc.store_compressed(
                    srcs_v.at[pl.ds(off, _LANES)], pv, mask=msk)
                off = off + jnp.sum(msk.astype(jnp.int32))

            pltpu.sync_copy(dt_hbm.at[b, pl.ds(s0, SB), :],
                            slab.at[pl.ds(0, SB), :])
            pltpu.async_copy(dt_hbm.at[b].at[srcs_v], rows_v, sem).wait()

            def apply16(jj, carry):
                dvec = dsts_v[pl.ds(jj * _LANES, _LANES)]
                for i in range(_LANES):
                    dst = dvec[i]
                    src = jj * _LANES + i
                    for cc in range(C // _LANES):
                        sl = pl.ds(cc * _LANES, _LANES)
                        slab[dst, sl] = slab[dst, sl] + rows_v[src, sl]
                return carry

            n16 = (off + _LANES - 1) // _LANES
            lax.fori_loop(0, n16, apply16, 0)

            pltpu.sync_copy(slab.at[pl.ds(0, SB), :],
                            out_hbm.at[b, pl.ds(s0, SB), :])

    return sc_kernel


def kernel(d, m, weight, poi_index):
    B, T, C, S = d.shape

    scale, data_t = pl.pallas_call(
        _scale_transpose_body,
        grid=(B,),
        in_specs=[
            pl.BlockSpec((1, 1, C, S), lambda i: (i, T - 1, 0, 0)),
            pl.BlockSpec((1, 1, C, S), lambda i: (i, T - 1, 0, 0)),
        ],
        out_specs=[
            pl.BlockSpec((1, S), lambda i: (0, 0)),
            pl.BlockSpec((1, S, C), lambda i: (i, 0, 0)),
        ],
        out_shape=[
            jax.ShapeDtypeStruct((1, S), jnp.float32),
            jax.ShapeDtypeStruct((B, S, C), jnp.float32),
        ],
    )(m, d)

    sc = _make_sc_kernel(B, T, C, S)
    out_t = sc(data_t, scale.reshape(S), poi_index.astype(jnp.int32))
    return out_t.reshape(B, _SIDE, _SIDE, C).transpose(0, 3, 1, 2)


# speedup vs baseline: 4.3020x; 2.0802x over previous
import functools

import jax
import jax.numpy as jnp
from jax import lax
from jax.experimental import pallas as pl
from jax.experimental.pallas import tpu as pltpu
from jax.experimental.pallas import tpu_sc as plsc

_SIDE = 32
_LANES = 16


def _scale_transpose_body(m_ref, d_ref, scale_ref, dt_ref):
    i = pl.program_id(0)
    part = jnp.sum(m_ref[0, 0], axis=0, keepdims=True)

    @pl.when(i == 0)
    def _():
        scale_ref[...] = part

    @pl.when(i > 0)
    def _():
        scale_ref[...] = scale_ref[...] + part

    @pl.when(i == pl.num_programs(0) - 1)
    def _():
        scale_ref[...] = jnp.where(scale_ref[...] == 0.0, 1.0, 0.0)

    dt_ref[0] = d_ref[0, 0].T


def _make_sc_kernel(B, T, C, S):
    mesh = plsc.VectorSubcoreMesh(core_axis_name="c", subcore_axis_name="s")
    info = plsc.get_sparse_core_info()
    nc, ns = info.num_cores, info.num_subcores
    nw = nc * ns
    halves = nw // B
    s_per_w = S // halves
    SB = 64
    NCH = s_per_w // SB
    DUMP = SB
    GB = _LANES

    @functools.partial(
        pl.kernel,
        out_type=jax.ShapeDtypeStruct((B, S, C), jnp.float32),
        mesh=mesh,
        compiler_params=pltpu.CompilerParams(
            needs_layout_passes=False, use_tc_tiling_on_sc=True),
        scratch_types=[
            pltpu.VMEM((S,), jnp.int32),
            pltpu.VMEM((S,), jnp.float32),
            pltpu.VMEM((2, SB + 8, C), jnp.float32),
            pltpu.VMEM((2, SB, C), jnp.float32),
            pltpu.VMEM((2, SB + _LANES), jnp.int32),
            pltpu.VMEM((2, SB), jnp.int32),
            pltpu.SemaphoreType.DMA,
            pltpu.SemaphoreType.DMA,
            pltpu.SemaphoreType.DMA,
            pltpu.SemaphoreType.DMA,
            pltpu.SemaphoreType.DMA,
            pltpu.SemaphoreType.DMA,
        ],
    )
    def sc_kernel(dt_hbm, scale_hbm, poi_hbm, out_hbm,
                  poi_v, scale_v, slab3, rows3, dsts2, srcs2,
                  si0, si1, sg0, sg1, so0, so1):
        wid = lax.axis_index("s") * nc + lax.axis_index("c")
        b = wid // halves
        s_base = (wid % halves) * s_per_w
        semin, semg, semout = (si0, si1), (sg0, sg1), (so0, so1)

        pltpu.sync_copy(poi_hbm, poi_v)
        pltpu.sync_copy(scale_hbm, scale_v)
        iota = lax.iota(jnp.int32, _LANES)
        cnts = [None] * NCH

        def in_dma(ci):
            p = ci % 2
            s0 = s_base + ci * SB
            return pltpu.make_async_copy(
                dt_hbm.at[b, pl.ds(s0, SB), :],
                slab3.at[p, pl.ds(0, SB), :], semin[p])

        def gather_dma(ci, k):
            p = ci % 2
            return pltpu.make_async_copy(
                dt_hbm.at[b].at[srcs2.at[p, pl.ds(k * GB, GB)]],
                rows3.at[p, pl.ds(k * GB, GB), :], semg[p])

        def out_dma(ci):
            p = ci % 2
            s0 = s_base + ci * SB
            return pltpu.make_async_copy(
                slab3.at[p, pl.ds(0, SB), :],
                out_hbm.at[b, pl.ds(s0, SB), :], semout[p])

        def start_chunk(ci):
            p = ci % 2
            s0 = s_base + ci * SB
            in_dma(ci).start()
            for t in range((SB + _LANES) // _LANES):
                dsts2[p, pl.ds(t * _LANES, _LANES)] = jnp.full(
                    (_LANES,), DUMP, jnp.int32)
            for t in range(SB // _LANES):
                srcs2[p, pl.ds(t * _LANES, _LANES)] = jnp.zeros(
                    (_LANES,), jnp.int32)
            off = jnp.int32(0)
            for k8 in range(SB // _LANES):
                sv = scale_v[pl.ds(s0 + k8 * _LANES, _LANES)]
                pv = poi_v[pl.ds(s0 + k8 * _LANES, _LANES)]
                msk = sv != 0.0
                plsc.store_compressed(
                    dsts2.at[p, pl.ds(off, _LANES)], iota + (k8 * _LANES),
                    mask=msk)
                plsc.store_compressed(
                    srcs2.at[p, pl.ds(off, _LANES)], pv, mask=msk)
                off = off + jnp.sum(msk.astype(jnp.int32))
            cnts[ci] = off
            for k in range(SB // GB):
                @pl.when(off > k * GB)
                def _(ci=ci, k=k):
                    gather_dma(ci, k).start()

        def finish_chunk(ci):
            p = ci % 2
            in_dma(ci).wait()
            off = cnts[ci]
            for k in range(SB // GB):
                @pl.when(off > k * GB)
                def _(ci=ci, k=k):
                    gather_dma(ci, k).wait()

            def apply16(jj, carry):
                dvec = dsts2[p, pl.ds(jj * _LANES, _LANES)]
                for i in range(_LANES):
                    dst = dvec[i]
                    src = jj * _LANES + i
                    for cc in range(C // _LANES):
                        sl = pl.ds(cc * _LANES, _LANES)
                        slab3[p, dst, sl] = (
                            slab3[p, dst, sl] + rows3[p, src, sl])
                return carry

            lax.fori_loop(0, (off + _LANES - 1) // _LANES, apply16, 0)
            out_dma(ci).start()

        start_chunk(0)
        for ci in range(NCH):
            if ci + 1 < NCH:
                if ci >= 1:
                    out_dma(ci - 1).wait()
                start_chunk(ci + 1)
            finish_chunk(ci)
        out_dma(NCH - 2).wait()
        out_dma(NCH - 1).wait()

    return sc_kernel


def kernel(d, m, weight, poi_index):
    B, T, C, S = d.shape

    scale, data_t = pl.pallas_call(
        _scale_transpose_body,
        grid=(B,),
        in_specs=[
            pl.BlockSpec((1, 1, C, S), lambda i: (i, T - 1, 0, 0)),
            pl.BlockSpec((1, 1, C, S), lambda i: (i, T - 1, 0, 0)),
        ],
        out_specs=[
            pl.BlockSpec((1, S), lambda i: (0, 0)),
            pl.BlockSpec((1, S, C), lambda i: (i, 0, 0)),
        ],
        out_shape=[
            jax.ShapeDtypeStruct((1, S), jnp.float32),
            jax.ShapeDtypeStruct((B, S, C), jnp.float32),
        ],
    )(m, d)

    sc = _make_sc_kernel(B, T, C, S)
    out_t = sc(data_t, scale.reshape(S), poi_index.astype(jnp.int32))
    return out_t.reshape(B, _SIDE, _SIDE, C).transpose(0, 3, 1, 2)


# speedup vs baseline: 4.3581x; 1.0130x over previous
import functools

import jax
import jax.numpy as jnp
from jax import lax
from jax.experimental import pallas as pl
from jax.experimental.pallas import tpu as pltpu
from jax.experimental.pallas import tpu_sc as plsc

_SIDE = 32
_LANES = 16


def _scale_transpose_body(m_ref, d_ref, scale_ref, dt_ref):
    i = pl.program_id(0)
    part = jnp.sum(m_ref[0, 0], axis=0, keepdims=True)

    @pl.when(i == 0)
    def _():
        scale_ref[...] = part

    @pl.when(i > 0)
    def _():
        scale_ref[...] = scale_ref[...] + part

    @pl.when(i == pl.num_programs(0) - 1)
    def _():
        scale_ref[...] = jnp.where(scale_ref[...] == 0.0, 1.0, 0.0)

    dt_ref[0] = d_ref[0, 0].T


def _make_sc_kernel(B, T, C, S):
    mesh = plsc.VectorSubcoreMesh(core_axis_name="c", subcore_axis_name="s")
    info = plsc.get_sparse_core_info()
    nc, ns = info.num_cores, info.num_subcores
    nw = nc * ns
    halves = nw // B
    s_per_w = S // halves
    SB = 64
    NCH = s_per_w // SB
    DUMP = SB
    GB = _LANES
    NBUF = 3

    @functools.partial(
        pl.kernel,
        out_type=jax.ShapeDtypeStruct((B, S, C), jnp.float32),
        mesh=mesh,
        compiler_params=pltpu.CompilerParams(
            needs_layout_passes=False, use_tc_tiling_on_sc=True),
        scratch_types=[
            pltpu.VMEM((S,), jnp.int32),
            pltpu.VMEM((S,), jnp.float32),
            pltpu.VMEM((NBUF, SB + 8, C), jnp.float32),
            pltpu.VMEM((NBUF, SB, C), jnp.float32),
            pltpu.VMEM((NBUF, SB + _LANES), jnp.int32),
            pltpu.VMEM((NBUF, SB), jnp.int32),
        ] + [pltpu.SemaphoreType.DMA] * (3 * NBUF),
    )
    def sc_kernel(dt_hbm, scale_hbm, poi_hbm, out_hbm,
                  poi_v, scale_v, slab3, rows3, dsts2, srcs2, *sems):
        wid = lax.axis_index("s") * nc + lax.axis_index("c")
        b = wid // halves
        s_base = (wid % halves) * s_per_w
        semin = sems[0:NBUF]
        semg = sems[NBUF:2 * NBUF]
        semout = sems[2 * NBUF:3 * NBUF]

        pltpu.sync_copy(poi_hbm, poi_v)
        pltpu.sync_copy(scale_hbm, scale_v)
        iota = lax.iota(jnp.int32, _LANES)
        cnts = [None] * NCH

        def in_dma(ci):
            p = ci % NBUF
            s0 = s_base + ci * SB
            return pltpu.make_async_copy(
                dt_hbm.at[b, pl.ds(s0, SB), :],
                slab3.at[p, pl.ds(0, SB), :], semin[p])

        def gather_dma(ci, k):
            p = ci % NBUF
            return pltpu.make_async_copy(
                dt_hbm.at[b].at[srcs2.at[p, pl.ds(k * GB, GB)]],
                rows3.at[p, pl.ds(k * GB, GB), :], semg[p])

        def out_dma(ci):
            p = ci % NBUF
            s0 = s_base + ci * SB
            return pltpu.make_async_copy(
                slab3.at[p, pl.ds(0, SB), :],
                out_hbm.at[b, pl.ds(s0, SB), :], semout[p])

        def start_chunk(ci):
            p = ci % NBUF
            s0 = s_base + ci * SB
            in_dma(ci).start()
            for t in range((SB + _LANES) // _LANES):
                dsts2[p, pl.ds(t * _LANES, _LANES)] = jnp.full(
                    (_LANES,), DUMP, jnp.int32)
            for t in range(SB // _LANES):
                srcs2[p, pl.ds(t * _LANES, _LANES)] = jnp.zeros(
                    (_LANES,), jnp.int32)
            off = jnp.int32(0)
            for k8 in range(SB // _LANES):
                sv = scale_v[pl.ds(s0 + k8 * _LANES, _LANES)]
                pv = poi_v[pl.ds(s0 + k8 * _LANES, _LANES)]
                msk = sv != 0.0
                plsc.store_compressed(
                    dsts2.at[p, pl.ds(off, _LANES)], iota + (k8 * _LANES),
                    mask=msk)
                plsc.store_compressed(
                    srcs2.at[p, pl.ds(off, _LANES)], pv, mask=msk)
                off = off + jnp.sum(msk.astype(jnp.int32))
            cnts[ci] = off
            for k in range(SB // GB):
                @pl.when(off > k * GB)
                def _(ci=ci, k=k):
                    gather_dma(ci, k).start()

        def finish_chunk(ci):
            p = ci % NBUF
            in_dma(ci).wait()
            off = cnts[ci]
            for k in range(SB // GB):
                @pl.when(off > k * GB)
                def _(ci=ci, k=k):
                    gather_dma(ci, k).wait()

            def apply16(jj, carry):
                dvec = dsts2[p, pl.ds(jj * _LANES, _LANES)]
                for i in range(_LANES):
                    dst = dvec[i]
                    src = jj * _LANES + i
                    for cc in range(C // _LANES):
                        sl = pl.ds(cc * _LANES, _LANES)
                        slab3[p, dst, sl] = (
                            slab3[p, dst, sl] + rows3[p, src, sl])
                return carry

            lax.fori_loop(0, (off + _LANES - 1) // _LANES, apply16, 0)
            out_dma(ci).start()

        start_chunk(0)
        for ci in range(NCH):
            if ci + 1 < NCH:
                if ci + 1 - NBUF >= 0:
                    out_dma(ci + 1 - NBUF).wait()
                start_chunk(ci + 1)
            finish_chunk(ci)
        for ci in range(max(0, NCH - NBUF), NCH):
            out_dma(ci).wait()

    return sc_kernel


def kernel(d, m, weight, poi_index):
    B, T, C, S = d.shape

    scale, data_t = pl.pallas_call(
        _scale_transpose_body,
        grid=(B,),
        in_specs=[
            pl.BlockSpec((1, 1, C, S), lambda i: (i, T - 1, 0, 0)),
            pl.BlockSpec((1, 1, C, S), lambda i: (i, T - 1, 0, 0)),
        ],
        out_specs=[
            pl.BlockSpec((1, S), lambda i: (0, 0)),
            pl.BlockSpec((1, S, C), lambda i: (i, 0, 0)),
        ],
        out_shape=[
            jax.ShapeDtypeStruct((1, S), jnp.float32),
            jax.ShapeDtypeStruct((B, S, C), jnp.float32),
        ],
    )(m, d)

    sc = _make_sc_kernel(B, T, C, S)
    out_t = sc(data_t, scale.reshape(S), poi_index.astype(jnp.int32))
    return out_t.reshape(B, _SIDE, _SIDE, C).transpose(0, 3, 1, 2)
